# Initial kernel scaffold; baseline (speedup 1.0000x reference)
import jax
import jax.numpy as jnp
from jax.experimental import pallas as pl


def kernel(in_feat, edge_index, W, b):
    raise NotImplementedError("write your pallas kernel here")


# speedup vs baseline: 2.5262x; 2.5262x over previous
import functools

import jax
import jax.numpy as jnp
from jax import lax
from jax.experimental import pallas as pl
from jax.experimental.pallas import tpu as pltpu
from jax.experimental.pallas import tpu_sc as plsc

NC = 2
NS = 16
NW = NC * NS

N = 10000
E = 320000
D = 128

NP = 10240
RPT = NP // NS
C = 128
NCHUNK = 80
EPT = NCHUNK * C
EPAD = NW * EPT


def _sc_degree_body(dst_hbm, zerosn_hbm, ones_hbm, hist_hbm,
                    dst_v, ones_v, hist_sh, sem):
    cid = lax.axis_index("c")
    sid = lax.axis_index("s")
    wid = cid * NS + sid
    pltpu.sync_copy(zerosn_hbm.at[pl.ds(sid * RPT, RPT)],
                    hist_sh.at[pl.ds(sid * RPT, RPT)])
    pltpu.sync_copy(ones_hbm, ones_v)
    pltpu.sync_copy(dst_hbm.at[pl.ds(wid * NCHUNK, NCHUNK)], dst_v)
    plsc.subcore_barrier()

    def body(j, carry):
        pltpu.sync_copy(ones_v, hist_sh.at[dst_v.at[j]], add=True)
        return carry

    lax.fori_loop(0, NCHUNK, body, 0)
    plsc.subcore_barrier()
    pltpu.sync_copy(hist_sh.at[pl.ds(sid * RPT, RPT)],
                    hist_hbm.at[cid].at[pl.ds(sid * RPT, RPT)])


@functools.lru_cache(maxsize=None)
def _sc_degree():
    mesh = plsc.VectorSubcoreMesh(core_axis_name="c", subcore_axis_name="s",
                                  num_cores=NC, num_subcores=NS)
    return pl.kernel(
        _sc_degree_body,
        out_type=jax.ShapeDtypeStruct((NC, NP), jnp.float32),
        mesh=mesh,
        scratch_types=[
            pltpu.VMEM((NCHUNK, C), jnp.int32),
            pltpu.VMEM((C,), jnp.float32),
            pltpu.VMEM_SHARED((NP,), jnp.float32),
            pltpu.SemaphoreType.DMA,
        ],
    )


def _sc_agg_body(x_hbm, src_hbm, dst_hbm, zeros_hbm, out_hbm,
                 src_v, dst_v, rows_v, acc_sh, sem):
    cid = lax.axis_index("c")
    sid = lax.axis_index("s")
    wid = cid * NS + sid
    pltpu.sync_copy(zeros_hbm.at[pl.ds(sid * RPT, RPT)],
                    acc_sh.at[pl.ds(sid * RPT, RPT)])
    pltpu.sync_copy(src_hbm.at[pl.ds(wid * NCHUNK, NCHUNK)], src_v)
    pltpu.sync_copy(dst_hbm.at[pl.ds(wid * NCHUNK, NCHUNK)], dst_v)
    plsc.subcore_barrier()

    def body(j, carry):
        pltpu.async_copy(x_hbm.at[src_v.at[j]], rows_v, sem).wait()
        pltpu.sync_copy(rows_v, acc_sh.at[dst_v.at[j]], add=True)
        return carry

    lax.fori_loop(0, NCHUNK, body, 0)
    plsc.subcore_barrier()
    pltpu.sync_copy(acc_sh.at[pl.ds(sid * RPT, RPT)],
                    out_hbm.at[cid].at[pl.ds(sid * RPT, RPT)])


@functools.lru_cache(maxsize=None)
def _sc_agg():
    mesh = plsc.VectorSubcoreMesh(core_axis_name="c", subcore_axis_name="s",
                                  num_cores=NC, num_subcores=NS)
    return pl.kernel(
        _sc_agg_body,
        out_type=jax.ShapeDtypeStruct((NC, NP, D), jnp.float32),
        mesh=mesh,
        scratch_types=[
            pltpu.VMEM((NCHUNK, C), jnp.int32),
            pltpu.VMEM((NCHUNK, C), jnp.int32),
            pltpu.VMEM((C, D), jnp.float32),
            pltpu.VMEM_SHARED((NP, D), jnp.float32),
            pltpu.SemaphoreType.DMA,
        ],
    )


def _norm_from_hist(hist):
    deg = hist[0] + hist[1]
    return lax.rsqrt(jnp.maximum(deg, 1.0))


def _tc_prep_body(hist_ref, x_ref, x1_ref):
    norm = _norm_from_hist(hist_ref[...])
    x1_ref[...] = x_ref[...] * norm[:, None]


def _tc_mid_body(hist_ref, agg_ref, x2_ref):
    norm = _norm_from_hist(hist_ref[...])
    agg = agg_ref[0] + agg_ref[1]
    x2_ref[...] = agg * (norm * norm)[:, None]


def _tc_final_body(hist_ref, agg_ref, w_ref, b_ref, out_ref):
    norm = _norm_from_hist(hist_ref[...])
    feat = (agg_ref[0] + agg_ref[1]) * norm[:, None]
    out_ref[...] = jnp.dot(feat, w_ref[...],
                           preferred_element_type=jnp.float32) + b_ref[...]


def kernel(in_feat, edge_index, W, b):
    npad = EPAD - E
    src = jnp.concatenate(
        [edge_index[0], jnp.zeros((npad,), jnp.int32)]).reshape(NW * NCHUNK, C)
    trash = N + (jnp.arange(npad, dtype=jnp.int32) % (NP - N))
    dst = jnp.concatenate([edge_index[1], trash]).reshape(NW * NCHUNK, C)

    x = jnp.pad(in_feat, ((0, NP - N), (0, 0)))
    zerosn = jnp.zeros((NP,), jnp.float32)
    ones = jnp.ones((C,), jnp.float32)
    zeros = jnp.zeros((NP, D), jnp.float32)

    hist = _sc_degree()(dst, zerosn, ones)

    x1 = pl.pallas_call(
        _tc_prep_body,
        out_shape=jax.ShapeDtypeStruct((NP, D), jnp.float32),
    )(hist, x)

    agg1 = _sc_agg()(x1, src, dst, zeros)

    x2 = pl.pallas_call(
        _tc_mid_body,
        out_shape=jax.ShapeDtypeStruct((NP, D), jnp.float32),
    )(hist, agg1)

    agg2 = _sc_agg()(x2, src, dst, zeros)

    out = pl.pallas_call(
        _tc_final_body,
        out_shape=jax.ShapeDtypeStruct((NP, D), jnp.float32),
    )(hist, agg2, W, b.reshape(1, D))

    return out[:N]


# speedup vs baseline: 2.9536x; 1.1692x over previous
import functools

import jax
import jax.numpy as jnp
from jax import lax
from jax.experimental import pallas as pl
from jax.experimental.pallas import tpu as pltpu
from jax.experimental.pallas import tpu_sc as plsc

NC = 2
NS = 16
NW = NC * NS

N = 10000
E = 320000
D = 128

NP = 10240
RPT = NP // NS
C = 80
NCHUNK = 128
EPT = NCHUNK * C
EPAD = NW * EPT


def _sc_degree_body(dst_hbm, zerosn_hbm, ones_hbm, hist_hbm,
                    dst_v, ones_v, hist_sh, sem):
    cid = lax.axis_index("c")
    sid = lax.axis_index("s")
    wid = cid * NS + sid
    pltpu.sync_copy(zerosn_hbm.at[pl.ds(sid * RPT, RPT)],
                    hist_sh.at[pl.ds(sid * RPT, RPT)])
    pltpu.sync_copy(ones_hbm, ones_v)
    pltpu.sync_copy(dst_hbm.at[pl.ds(wid * NCHUNK, NCHUNK)], dst_v)
    plsc.subcore_barrier()

    def body(j, carry):
        pltpu.sync_copy(ones_v, hist_sh.at[dst_v.at[j]], add=True)
        return carry

    lax.fori_loop(0, NCHUNK, body, 0)
    plsc.subcore_barrier()
    pltpu.sync_copy(hist_sh.at[pl.ds(sid * RPT, RPT)],
                    hist_hbm.at[cid].at[pl.ds(sid * RPT, RPT)])


@functools.lru_cache(maxsize=None)
def _sc_degree():
    mesh = plsc.VectorSubcoreMesh(core_axis_name="c", subcore_axis_name="s",
                                  num_cores=NC, num_subcores=NS)
    return pl.kernel(
        _sc_degree_body,
        out_type=jax.ShapeDtypeStruct((NC, NP), jnp.float32),
        mesh=mesh,
        scratch_types=[
            pltpu.VMEM((NCHUNK, C), jnp.int32),
            pltpu.VMEM((C,), jnp.float32),
            pltpu.VMEM_SHARED((NP,), jnp.float32),
            pltpu.SemaphoreType.DMA,
        ],
    )


NBUF = 4
LOOK = 2
BS = 8
NBLK = NCHUNK // BS
SB = 3
DB = 4


def _sc_agg_body(x_hbm, src_hbm, dst_hbm, zeros_hbm, out_hbm,
                 srcb_v, dstb_v, rows_v, acc_sh, *sems):
    gsem = sems[:NBUF]
    ssem = sems[NBUF:]
    cid = lax.axis_index("c")
    sid = lax.axis_index("s")
    wid = cid * NS + sid
    pltpu.sync_copy(zeros_hbm.at[pl.ds(sid * RPT, RPT)],
                    acc_sh.at[pl.ds(sid * RPT, RPT)])
    for b in range(2):
        pltpu.sync_copy(src_hbm.at[pl.ds(wid * NCHUNK + b * BS, BS)],
                        srcb_v.at[b])
        pltpu.sync_copy(dst_hbm.at[pl.ds(wid * NCHUNK + b * BS, BS)],
                        dstb_v.at[b])
    plsc.subcore_barrier()

    for k in range(LOOK):
        pltpu.async_copy(x_hbm.at[srcb_v.at[0].at[k]], rows_v.at[k], gsem[k])

    def body(g, carry):
        for b in range(NBUF):
            j = g * NBUF + b
            blk = lax.div(j, BS)
            row = lax.rem(j, BS)
            pltpu.make_async_copy(
                x_hbm.at[srcb_v.at[lax.rem(blk, SB)].at[row]],
                rows_v.at[b], gsem[b]).wait()
            pltpu.async_copy(rows_v.at[b],
                             acc_sh.at[dstb_v.at[lax.rem(blk, DB)].at[row]],
                             ssem[b], add=True)

            @pl.when((row == 0) & (blk + 2 < NBLK))
            def _():
                pltpu.sync_copy(
                    src_hbm.at[pl.ds(wid * NCHUNK + (blk + 2) * BS, BS)],
                    srcb_v.at[lax.rem(blk + 2, SB)])
                pltpu.sync_copy(
                    dst_hbm.at[pl.ds(wid * NCHUNK + (blk + 2) * BS, BS)],
                    dstb_v.at[lax.rem(blk + 2, DB)])

            bn = (b + LOOK) % NBUF
            jn = j + LOOK

            @pl.when(jn < NCHUNK)
            def _():
                @pl.when(jn >= NBUF)
                def _():
                    pltpu.make_async_copy(rows_v.at[bn],
                                          acc_sh.at[dstb_v.at[0].at[0]],
                                          ssem[bn]).wait()

                pltpu.async_copy(
                    x_hbm.at[srcb_v.at[lax.rem(lax.div(jn, BS), SB)]
                             .at[lax.rem(jn, BS)]],
                    rows_v.at[bn], gsem[bn])

        return carry

    lax.fori_loop(0, NCHUNK // NBUF, body, 0)
    for b in range(NBUF):
        pltpu.make_async_copy(rows_v.at[b], acc_sh.at[dstb_v.at[0].at[0]],
                              ssem[b]).wait()
    plsc.subcore_barrier()
    pltpu.sync_copy(acc_sh.at[pl.ds(sid * RPT, RPT)],
                    out_hbm.at[cid].at[pl.ds(sid * RPT, RPT)])


@functools.lru_cache(maxsize=None)
def _sc_agg():
    mesh = plsc.VectorSubcoreMesh(core_axis_name="c", subcore_axis_name="s",
                                  num_cores=NC, num_subcores=NS)
    return pl.kernel(
        _sc_agg_body,
        out_type=jax.ShapeDtypeStruct((NC, NP, D), jnp.float32),
        mesh=mesh,
        scratch_types=[
            pltpu.VMEM((SB, BS, C), jnp.int32),
            pltpu.VMEM((DB, BS, C), jnp.int32),
            pltpu.VMEM((NBUF, C, D), jnp.float32),
            pltpu.VMEM_SHARED((NP, D), jnp.float32),
        ] + [pltpu.SemaphoreType.DMA] * (2 * NBUF),
    )


def _norm_from_hist(hist):
    deg = hist[0] + hist[1]
    return lax.rsqrt(jnp.maximum(deg, 1.0))


def _tc_prep_body(hist_ref, x_ref, x1_ref):
    norm = _norm_from_hist(hist_ref[...])
    x1_ref[...] = x_ref[...] * norm[:, None]


def _tc_mid_body(hist_ref, agg_ref, x2_ref):
    norm = _norm_from_hist(hist_ref[...])
    agg = agg_ref[0] + agg_ref[1]
    x2_ref[...] = agg * (norm * norm)[:, None]


def _tc_final_body(hist_ref, agg_ref, w_ref, b_ref, out_ref):
    norm = _norm_from_hist(hist_ref[...])
    feat = (agg_ref[0] + agg_ref[1]) * norm[:, None]
    out_ref[...] = jnp.dot(feat, w_ref[...],
                           preferred_element_type=jnp.float32) + b_ref[...]


def kernel(in_feat, edge_index, W, b):
    npad = EPAD - E
    src = jnp.concatenate(
        [edge_index[0], jnp.zeros((npad,), jnp.int32)]).reshape(NW * NCHUNK, C)
    trash = N + (jnp.arange(npad, dtype=jnp.int32) % (NP - N))
    dst = jnp.concatenate([edge_index[1], trash]).reshape(NW * NCHUNK, C)

    x = jnp.pad(in_feat, ((0, NP - N), (0, 0)))
    zerosn = jnp.zeros((NP,), jnp.float32)
    ones = jnp.ones((C,), jnp.float32)
    zeros = jnp.zeros((NP, D), jnp.float32)

    hist = _sc_degree()(dst, zerosn, ones)

    x1 = pl.pallas_call(
        _tc_prep_body,
        out_shape=jax.ShapeDtypeStruct((NP, D), jnp.float32),
    )(hist, x)

    agg1 = _sc_agg()(x1, src, dst, zeros)

    x2 = pl.pallas_call(
        _tc_mid_body,
        out_shape=jax.ShapeDtypeStruct((NP, D), jnp.float32),
    )(hist, agg1)

    agg2 = _sc_agg()(x2, src, dst, zeros)

    out = pl.pallas_call(
        _tc_final_body,
        out_shape=jax.ShapeDtypeStruct((NP, D), jnp.float32),
    )(hist, agg2, W, b.reshape(1, D))

    return out[:N]


# speedup vs baseline: 3.4761x; 1.1769x over previous
import functools

import jax
import jax.numpy as jnp
from jax import lax
from jax.experimental import pallas as pl
from jax.experimental.pallas import tpu as pltpu
from jax.experimental.pallas import tpu_sc as plsc

NC = 2
NS = 16
NW = NC * NS

N = 10000
E = 320000
D = 128

NP = 10240
RPT = NP // NS
C = 64
NCHUNK = 160
EPT = NCHUNK * C
EPAD = NW * EPT


def _sc_degree_body(dst_hbm, zerosn_hbm, ones_hbm, hist_hbm,
                    dst_v, ones_v, hist_sh, sem):
    cid = lax.axis_index("c")
    sid = lax.axis_index("s")
    wid = cid * NS + sid
    pltpu.sync_copy(zerosn_hbm.at[pl.ds(sid * RPT, RPT)],
                    hist_sh.at[pl.ds(sid * RPT, RPT)])
    pltpu.sync_copy(ones_hbm, ones_v)
    pltpu.sync_copy(dst_hbm.at[pl.ds(wid * NCHUNK, NCHUNK)], dst_v)
    plsc.subcore_barrier()

    def body(j, carry):
        pltpu.sync_copy(ones_v, hist_sh.at[dst_v.at[j]], add=True)
        return carry

    lax.fori_loop(0, NCHUNK, body, 0)
    plsc.subcore_barrier()
    pltpu.sync_copy(hist_sh.at[pl.ds(sid * RPT, RPT)],
                    hist_hbm.at[cid].at[pl.ds(sid * RPT, RPT)])


@functools.lru_cache(maxsize=None)
def _sc_degree():
    mesh = plsc.VectorSubcoreMesh(core_axis_name="c", subcore_axis_name="s",
                                  num_cores=NC, num_subcores=NS)
    return pl.kernel(
        _sc_degree_body,
        out_type=jax.ShapeDtypeStruct((NC, NP), jnp.float32),
        mesh=mesh,
        scratch_types=[
            pltpu.VMEM((NCHUNK, C), jnp.int32),
            pltpu.VMEM((C,), jnp.float32),
            pltpu.VMEM_SHARED((NP,), jnp.float32),
            pltpu.SemaphoreType.DMA,
        ],
    )


NBUF = 5
LOOK = 2
BS = 8
NBLK = NCHUNK // BS
SB = 3
DB = 4


def _sc_agg_body(x_hbm, src_hbm, dst_hbm, zeros_hbm, out_hbm,
                 srcb_v, dstb_v, rows_v, acc_sh, *sems):
    gsem = sems[:NBUF]
    ssem = sems[NBUF:]
    cid = lax.axis_index("c")
    sid = lax.axis_index("s")
    wid = cid * NS + sid
    pltpu.sync_copy(zeros_hbm.at[pl.ds(sid * RPT, RPT)],
                    acc_sh.at[pl.ds(sid * RPT, RPT)])
    for b in range(2):
        pltpu.sync_copy(src_hbm.at[pl.ds(wid * NCHUNK + b * BS, BS)],
                        srcb_v.at[b])
        pltpu.sync_copy(dst_hbm.at[pl.ds(wid * NCHUNK + b * BS, BS)],
                        dstb_v.at[b])
    plsc.subcore_barrier()

    for k in range(LOOK):
        pltpu.async_copy(x_hbm.at[srcb_v.at[0].at[k]], rows_v.at[k], gsem[k])

    def body(g, carry):
        for b in range(NBUF):
            j = g * NBUF + b
            blk = lax.div(j, BS)
            row = lax.rem(j, BS)
            pltpu.make_async_copy(
                x_hbm.at[srcb_v.at[lax.rem(blk, SB)].at[row]],
                rows_v.at[b], gsem[b]).wait()
            pltpu.async_copy(rows_v.at[b],
                             acc_sh.at[dstb_v.at[lax.rem(blk, DB)].at[row]],
                             ssem[b], add=True)

            @pl.when((row == 0) & (blk + 2 < NBLK))
            def _():
                pltpu.sync_copy(
                    src_hbm.at[pl.ds(wid * NCHUNK + (blk + 2) * BS, BS)],
                    srcb_v.at[lax.rem(blk + 2, SB)])
                pltpu.sync_copy(
                    dst_hbm.at[pl.ds(wid * NCHUNK + (blk + 2) * BS, BS)],
                    dstb_v.at[lax.rem(blk + 2, DB)])

            bn = (b + LOOK) % NBUF
            jn = j + LOOK

            @pl.when(jn < NCHUNK)
            def _():
                @pl.when(jn >= NBUF)
                def _():
                    pltpu.make_async_copy(rows_v.at[bn],
                                          acc_sh.at[dstb_v.at[0].at[0]],
                                          ssem[bn]).wait()

                pltpu.async_copy(
                    x_hbm.at[srcb_v.at[lax.rem(lax.div(jn, BS), SB)]
                             .at[lax.rem(jn, BS)]],
                    rows_v.at[bn], gsem[bn])

        return carry

    lax.fori_loop(0, NCHUNK // NBUF, body, 0)
    for b in range(NBUF):
        pltpu.make_async_copy(rows_v.at[b], acc_sh.at[dstb_v.at[0].at[0]],
                              ssem[b]).wait()
    plsc.subcore_barrier()
    pltpu.sync_copy(acc_sh.at[pl.ds(sid * RPT, RPT)],
                    out_hbm.at[cid].at[pl.ds(sid * RPT, RPT)])


@functools.lru_cache(maxsize=None)
def _sc_agg():
    mesh = plsc.VectorSubcoreMesh(core_axis_name="c", subcore_axis_name="s",
                                  num_cores=NC, num_subcores=NS)
    return pl.kernel(
        _sc_agg_body,
        out_type=jax.ShapeDtypeStruct((NC, NP, D), jnp.float32),
        mesh=mesh,
        scratch_types=[
            pltpu.VMEM((SB, BS, C), jnp.int32),
            pltpu.VMEM((DB, BS, C), jnp.int32),
            pltpu.VMEM((NBUF, C, D), jnp.float32),
            pltpu.VMEM_SHARED((NP, D), jnp.float32),
        ] + [pltpu.SemaphoreType.DMA] * (2 * NBUF),
    )


def _norm_from_hist(hist):
    deg = hist[0] + hist[1]
    return lax.rsqrt(jnp.maximum(deg, 1.0))


def _tc_prep_body(hist_ref, x_ref, x1_ref):
    norm = _norm_from_hist(hist_ref[...])
    x1_ref[...] = x_ref[...] * norm[:, None]


def _tc_mid_body(hist_ref, agg_ref, x2_ref):
    norm = _norm_from_hist(hist_ref[...])
    agg = agg_ref[0] + agg_ref[1]
    x2_ref[...] = agg * (norm * norm)[:, None]


def _tc_final_body(hist_ref, agg_ref, w_ref, b_ref, out_ref):
    norm = _norm_from_hist(hist_ref[...])
    feat = (agg_ref[0] + agg_ref[1]) * norm[:, None]
    out_ref[...] = jnp.dot(feat, w_ref[...],
                           preferred_element_type=jnp.float32) + b_ref[...]


def kernel(in_feat, edge_index, W, b):
    npad = EPAD - E
    src = jnp.concatenate(
        [edge_index[0], jnp.zeros((npad,), jnp.int32)]).reshape(NW * NCHUNK, C)
    trash = N + (jnp.arange(npad, dtype=jnp.int32) % (NP - N))
    dst = jnp.concatenate([edge_index[1], trash]).reshape(NW * NCHUNK, C)

    x = jnp.pad(in_feat, ((0, NP - N), (0, 0)))
    zerosn = jnp.zeros((NP,), jnp.float32)
    ones = jnp.ones((C,), jnp.float32)
    zeros = jnp.zeros((NP, D), jnp.float32)

    hist = _sc_degree()(dst, zerosn, ones)

    x1 = pl.pallas_call(
        _tc_prep_body,
        out_shape=jax.ShapeDtypeStruct((NP, D), jnp.float32),
    )(hist, x)

    agg1 = _sc_agg()(x1, src, dst, zeros)

    x2 = pl.pallas_call(
        _tc_mid_body,
        out_shape=jax.ShapeDtypeStruct((NP, D), jnp.float32),
    )(hist, agg1)

    agg2 = _sc_agg()(x2, src, dst, zeros)

    out = pl.pallas_call(
        _tc_final_body,
        out_shape=jax.ShapeDtypeStruct((NP, D), jnp.float32),
    )(hist, agg2, W, b.reshape(1, D))

    return out[:N]


# speedup vs baseline: 3.7862x; 1.0892x over previous
import functools

import jax
import jax.numpy as jnp
from jax import lax
from jax.experimental import pallas as pl
from jax.experimental.pallas import tpu as pltpu
from jax.experimental.pallas import tpu_sc as plsc

NC = 2
NS = 16
NW = NC * NS

N = 10000
E = 320000
D = 128

NP = 10240
RPT = NP // NS
C = 64
NCHUNK = 160
EPT = NCHUNK * C
EPAD = NW * EPT


def _sc_degree_body(dst_hbm, zerosn_hbm, ones_hbm, hist_hbm,
                    dst_v, ones_v, hist_sh, sem):
    cid = lax.axis_index("c")
    sid = lax.axis_index("s")
    wid = cid * NS + sid
    pltpu.sync_copy(zerosn_hbm.at[pl.ds(sid * RPT, RPT)],
                    hist_sh.at[pl.ds(sid * RPT, RPT)])
    pltpu.sync_copy(ones_hbm, ones_v)
    pltpu.sync_copy(dst_hbm.at[pl.ds(wid * NCHUNK, NCHUNK)], dst_v)
    plsc.subcore_barrier()

    def body(j, carry):
        pltpu.sync_copy(ones_v, hist_sh.at[dst_v.at[j]], add=True)
        return carry

    lax.fori_loop(0, NCHUNK, body, 0)
    plsc.subcore_barrier()
    pltpu.sync_copy(hist_sh.at[pl.ds(sid * RPT, RPT)],
                    hist_hbm.at[cid].at[pl.ds(sid * RPT, RPT)])


@functools.lru_cache(maxsize=None)
def _sc_degree():
    mesh = plsc.VectorSubcoreMesh(core_axis_name="c", subcore_axis_name="s",
                                  num_cores=NC, num_subcores=NS)
    return pl.kernel(
        _sc_degree_body,
        out_type=jax.ShapeDtypeStruct((NC, NP), jnp.float32),
        mesh=mesh,
        scratch_types=[
            pltpu.VMEM((NCHUNK, C), jnp.int32),
            pltpu.VMEM((C,), jnp.float32),
            pltpu.VMEM_SHARED((NP,), jnp.float32),
            pltpu.SemaphoreType.DMA,
        ],
    )


NBUF = 5
LOOK = 2
BS = 8
SB = 3
DB = 4
Q0 = 240
Q1 = 2 * NCHUNK - Q0


def _sc_agg_body(x_hbm, src_hbm, dst_hbm, zeros_hbm, out_hbm,
                 srcb_v, dstb_v, rows_v, acc_sh, *sems):
    gsem = sems[:NBUF]
    ssem = sems[NBUF:]
    cid = lax.axis_index("c")
    sid = lax.axis_index("s")
    qc = lax.select(cid == 0, Q0, Q1)
    base = lax.select(cid == 0, sid * Q0, NS * Q0 + sid * Q1)
    pltpu.sync_copy(zeros_hbm.at[pl.ds(sid * RPT, RPT)],
                    acc_sh.at[pl.ds(sid * RPT, RPT)])
    for b in range(2):
        pltpu.sync_copy(src_hbm.at[pl.ds(base + b * BS, BS)],
                        srcb_v.at[b])
        pltpu.sync_copy(dst_hbm.at[pl.ds(base + b * BS, BS)],
                        dstb_v.at[b])
    plsc.subcore_barrier()

    for k in range(LOOK):
        pltpu.async_copy(x_hbm.at[srcb_v.at[0].at[k]], rows_v.at[k], gsem[k])

    def body(g, carry):
        for b in range(NBUF):
            j = g * NBUF + b
            blk = lax.div(j, BS)
            row = lax.rem(j, BS)
            pltpu.make_async_copy(
                x_hbm.at[srcb_v.at[lax.rem(blk, SB)].at[row]],
                rows_v.at[b], gsem[b]).wait()
            pltpu.async_copy(rows_v.at[b],
                             acc_sh.at[dstb_v.at[lax.rem(blk, DB)].at[row]],
                             ssem[b], add=True)

            @pl.when((row == 0) & ((blk + 2) * BS < qc))
            def _():
                pltpu.sync_copy(
                    src_hbm.at[pl.ds(base + (blk + 2) * BS, BS)],
                    srcb_v.at[lax.rem(blk + 2, SB)])
                pltpu.sync_copy(
                    dst_hbm.at[pl.ds(base + (blk + 2) * BS, BS)],
                    dstb_v.at[lax.rem(blk + 2, DB)])

            bn = (b + LOOK) % NBUF
            jn = j + LOOK

            @pl.when(jn < qc)
            def _():
                @pl.when(jn >= NBUF)
                def _():
                    pltpu.make_async_copy(rows_v.at[bn],
                                          acc_sh.at[dstb_v.at[0].at[0]],
                                          ssem[bn]).wait()

                pltpu.async_copy(
                    x_hbm.at[srcb_v.at[lax.rem(lax.div(jn, BS), SB)]
                             .at[lax.rem(jn, BS)]],
                    rows_v.at[bn], gsem[bn])

        return carry

    lax.fori_loop(0, lax.div(qc, NBUF), body, 0)
    for b in range(NBUF):
        pltpu.make_async_copy(rows_v.at[b], acc_sh.at[dstb_v.at[0].at[0]],
                              ssem[b]).wait()
    plsc.subcore_barrier()
    pltpu.sync_copy(acc_sh.at[pl.ds(sid * RPT, RPT)],
                    out_hbm.at[cid].at[pl.ds(sid * RPT, RPT)])


@functools.lru_cache(maxsize=None)
def _sc_agg():
    mesh = plsc.VectorSubcoreMesh(core_axis_name="c", subcore_axis_name="s",
                                  num_cores=NC, num_subcores=NS)
    return pl.kernel(
        _sc_agg_body,
        out_type=jax.ShapeDtypeStruct((NC, NP, D), jnp.float32),
        mesh=mesh,
        scratch_types=[
            pltpu.VMEM((SB, BS, C), jnp.int32),
            pltpu.VMEM((DB, BS, C), jnp.int32),
            pltpu.VMEM((NBUF, C, D), jnp.float32),
            pltpu.VMEM_SHARED((NP, D), jnp.float32),
        ] + [pltpu.SemaphoreType.DMA] * (2 * NBUF),
    )


def _norm_from_hist(hist):
    deg = hist[0] + hist[1]
    return lax.rsqrt(jnp.maximum(deg, 1.0))


def _tc_prep_body(hist_ref, x_ref, x1_ref):
    norm = _norm_from_hist(hist_ref[...])
    x1_ref[...] = x_ref[...] * norm[:, None]


def _tc_mid_body(hist_ref, agg_ref, x2_ref):
    norm = _norm_from_hist(hist_ref[...])
    agg = agg_ref[0] + agg_ref[1]
    x2_ref[...] = agg * (norm * norm)[:, None]


def _tc_final_body(hist_ref, agg_ref, w_ref, b_ref, out_ref):
    norm = _norm_from_hist(hist_ref[...])
    feat = (agg_ref[0] + agg_ref[1]) * norm[:, None]
    out_ref[...] = jnp.dot(feat, w_ref[...],
                           preferred_element_type=jnp.float32) + b_ref[...]


def kernel(in_feat, edge_index, W, b):
    npad = EPAD - E
    src = jnp.concatenate(
        [edge_index[0], jnp.zeros((npad,), jnp.int32)]).reshape(NW * NCHUNK, C)
    trash = N + (jnp.arange(npad, dtype=jnp.int32) % (NP - N))
    dst = jnp.concatenate([edge_index[1], trash]).reshape(NW * NCHUNK, C)

    x = jnp.pad(in_feat, ((0, NP - N), (0, 0)))
    zerosn = jnp.zeros((NP,), jnp.float32)
    ones = jnp.ones((C,), jnp.float32)
    zeros = jnp.zeros((NP, D), jnp.float32)

    hist = _sc_degree()(dst, zerosn, ones)

    x1 = pl.pallas_call(
        _tc_prep_body,
        out_shape=jax.ShapeDtypeStruct((NP, D), jnp.float32),
    )(hist, x)

    agg1 = _sc_agg()(x1, src, dst, zeros)

    x2 = pl.pallas_call(
        _tc_mid_body,
        out_shape=jax.ShapeDtypeStruct((NP, D), jnp.float32),
    )(hist, agg1)

    agg2 = _sc_agg()(x2, src, dst, zeros)

    out = pl.pallas_call(
        _tc_final_body,
        out_shape=jax.ShapeDtypeStruct((NP, D), jnp.float32),
    )(hist, agg2, W, b.reshape(1, D))

    return out[:N]


# speedup vs baseline: 9.2045x; 2.4311x over previous
import functools

import jax
import jax.numpy as jnp
from jax import lax
from jax.experimental import pallas as pl
from jax.experimental.pallas import tpu as pltpu
from jax.experimental.pallas import tpu_sc as plsc

NC = 2
NS = 16
NW = NC * NS

N = 10000
E = 320000
D = 128

NP = 10240
RPT = NP // NS
C = 64
NCHUNK = 160
EPT = NCHUNK * C
EPAD = NW * EPT


def _sc_degree_body(dst_hbm, zerosn_hbm, ones_hbm, hist_hbm,
                    dst_v, ones_v, hist_sh, sem):
    cid = lax.axis_index("c")
    sid = lax.axis_index("s")
    wid = cid * NS + sid
    pltpu.sync_copy(zerosn_hbm.at[pl.ds(sid * RPT, RPT)],
                    hist_sh.at[pl.ds(sid * RPT, RPT)])
    pltpu.sync_copy(ones_hbm, ones_v)
    pltpu.sync_copy(dst_hbm.at[pl.ds(wid * NCHUNK, NCHUNK)], dst_v)
    plsc.subcore_barrier()

    def body(j, carry):
        pltpu.sync_copy(ones_v, hist_sh.at[dst_v.at[j]], add=True)
        return carry

    lax.fori_loop(0, NCHUNK, body, 0)
    plsc.subcore_barrier()
    pltpu.sync_copy(hist_sh.at[pl.ds(sid * RPT, RPT)],
                    hist_hbm.at[cid].at[pl.ds(sid * RPT, RPT)])


@functools.lru_cache(maxsize=None)
def _sc_degree():
    mesh = plsc.VectorSubcoreMesh(core_axis_name="c", subcore_axis_name="s",
                                  num_cores=NC, num_subcores=NS)
    return pl.kernel(
        _sc_degree_body,
        out_type=jax.ShapeDtypeStruct((NC, NP), jnp.float32),
        mesh=mesh,
        scratch_types=[
            pltpu.VMEM((NCHUNK, C), jnp.int32),
            pltpu.VMEM((C,), jnp.float32),
            pltpu.VMEM_SHARED((NP,), jnp.float32),
            pltpu.SemaphoreType.DMA,
        ],
    )


NBUF = 5
LOOK = 2
BS = 8
SB = 3
DB = 4
Q0 = 160
Q1 = 2 * NCHUNK - Q0


def _sc_agg_body(x_hbm, src_hbm, dst_hbm, zeros_hbm, out_hbm,
                 srcb_v, dstb_v, rows_v, acc_sh, *sems):
    gsem = sems[:NBUF]
    ssem = sems[NBUF:]
    cid = lax.axis_index("c")
    sid = lax.axis_index("s")
    qc = lax.select(cid == 0, Q0, Q1)
    base = lax.select(cid == 0, sid * Q0, NS * Q0 + sid * Q1)
    pltpu.sync_copy(zeros_hbm.at[pl.ds(sid * RPT, RPT)],
                    acc_sh.at[pl.ds(sid * RPT, RPT)])
    for b in range(2):
        pltpu.sync_copy(src_hbm.at[pl.ds(base + b * BS, BS)],
                        srcb_v.at[b])
        pltpu.sync_copy(dst_hbm.at[pl.ds(base + b * BS, BS)],
                        dstb_v.at[b])
    plsc.subcore_barrier()

    for k in range(LOOK):
        pltpu.async_copy(x_hbm.at[srcb_v.at[0].at[k]], rows_v.at[k], gsem[k])

    def body(g, carry):
        for b in range(NBUF):
            j = g * NBUF + b
            blk = lax.div(j, BS)
            row = lax.rem(j, BS)
            pltpu.make_async_copy(
                x_hbm.at[srcb_v.at[lax.rem(blk, SB)].at[row]],
                rows_v.at[b], gsem[b]).wait()
            pltpu.async_copy(rows_v.at[b],
                             acc_sh.at[dstb_v.at[lax.rem(blk, DB)].at[row]],
                             ssem[b], add=True)

            @pl.when((row == 0) & ((blk + 2) * BS < qc))
            def _():
                pltpu.sync_copy(
                    src_hbm.at[pl.ds(base + (blk + 2) * BS, BS)],
                    srcb_v.at[lax.rem(blk + 2, SB)])
                pltpu.sync_copy(
                    dst_hbm.at[pl.ds(base + (blk + 2) * BS, BS)],
                    dstb_v.at[lax.rem(blk + 2, DB)])

            bn = (b + LOOK) % NBUF
            jn = j + LOOK

            @pl.when(jn < qc)
            def _():
                @pl.when(jn >= NBUF)
                def _():
                    pltpu.make_async_copy(rows_v.at[bn],
                                          acc_sh.at[dstb_v.at[0].at[0]],
                                          ssem[bn]).wait()

                pltpu.async_copy(
                    x_hbm.at[srcb_v.at[lax.rem(lax.div(jn, BS), SB)]
                             .at[lax.rem(jn, BS)]],
                    rows_v.at[bn], gsem[bn])

        return carry

    lax.fori_loop(0, lax.div(qc, NBUF), body, 0)
    for b in range(NBUF):
        pltpu.make_async_copy(rows_v.at[b], acc_sh.at[dstb_v.at[0].at[0]],
                              ssem[b]).wait()
    plsc.subcore_barrier()
    pltpu.sync_copy(acc_sh.at[pl.ds(sid * RPT, RPT)],
                    out_hbm.at[cid].at[pl.ds(sid * RPT, RPT)])


@functools.lru_cache(maxsize=None)
def _sc_agg():
    mesh = plsc.VectorSubcoreMesh(core_axis_name="c", subcore_axis_name="s",
                                  num_cores=NC, num_subcores=NS)
    return pl.kernel(
        _sc_agg_body,
        out_type=jax.ShapeDtypeStruct((NC, NP, D), jnp.float32),
        mesh=mesh,
        scratch_types=[
            pltpu.VMEM((SB, BS, C), jnp.int32),
            pltpu.VMEM((DB, BS, C), jnp.int32),
            pltpu.VMEM((NBUF, C, D), jnp.float32),
            pltpu.VMEM_SHARED((NP, D), jnp.float32),
        ] + [pltpu.SemaphoreType.DMA] * (2 * NBUF),
    )


def _norm_from_hist(hist):
    deg = hist[0] + hist[1]
    return lax.rsqrt(jnp.maximum(deg, 1.0))


def _tc_prep_body(hist_ref, x_ref, x1_ref):
    norm = _norm_from_hist(hist_ref[...])
    x1_ref[...] = x_ref[...] * norm[:, None]


def _tc_mid_body(hist_ref, agg_ref, x2_ref):
    norm = _norm_from_hist(hist_ref[...])
    agg = agg_ref[0] + agg_ref[1]
    x2_ref[...] = agg * (norm * norm)[:, None]


def _tc_final_body(hist_ref, agg_ref, w_ref, b_ref, out_ref):
    norm = _norm_from_hist(hist_ref[...])
    feat = (agg_ref[0] + agg_ref[1]) * norm[:, None]
    out_ref[...] = jnp.dot(feat, w_ref[...],
                           preferred_element_type=jnp.float32) + b_ref[...]


def kernel(in_feat, edge_index, W, b):
    npad = EPAD - E
    padsrc = jnp.arange(npad, dtype=jnp.int32) % N
    src = jnp.concatenate(
        [edge_index[0], padsrc]).reshape(NW * NCHUNK, C)
    trash = N + (jnp.arange(npad, dtype=jnp.int32) % (NP - N))
    dst = jnp.concatenate([edge_index[1], trash]).reshape(NW * NCHUNK, C)

    x = jnp.pad(in_feat, ((0, NP - N), (0, 0)))
    zerosn = jnp.zeros((NP,), jnp.float32)
    ones = jnp.ones((C,), jnp.float32)
    zeros = jnp.zeros((NP, D), jnp.float32)

    hist = _sc_degree()(dst, zerosn, ones)

    x1 = pl.pallas_call(
        _tc_prep_body,
        out_shape=jax.ShapeDtypeStruct((NP, D), jnp.float32),
    )(hist, x)

    agg1 = _sc_agg()(x1, src, dst, zeros)

    x2 = pl.pallas_call(
        _tc_mid_body,
        out_shape=jax.ShapeDtypeStruct((NP, D), jnp.float32),
    )(hist, agg1)

    agg2 = _sc_agg()(x2, src, dst, zeros)

    out = pl.pallas_call(
        _tc_final_body,
        out_shape=jax.ShapeDtypeStruct((NP, D), jnp.float32),
    )(hist, agg2, W, b.reshape(1, D))

    return out[:N]


# speedup vs baseline: 9.7801x; 1.0625x over previous
import functools

import jax
import jax.numpy as jnp
from jax import lax
from jax.experimental import pallas as pl
from jax.experimental.pallas import tpu as pltpu
from jax.experimental.pallas import tpu_sc as plsc

NC = 2
NS = 16
NW = NC * NS

N = 10000
E = 320000
D = 128

NP = 10240
RPT = NP // NS
C = 80
NCHUNK = 128
EPT = NCHUNK * C
EPAD = NW * EPT


def _sc_degree_body(dst_hbm, zerosn_hbm, ones_hbm, hist_hbm,
                    dst_v, ones_v, hist_sh, sem):
    cid = lax.axis_index("c")
    sid = lax.axis_index("s")
    wid = cid * NS + sid
    pltpu.sync_copy(zerosn_hbm.at[pl.ds(sid * RPT, RPT)],
                    hist_sh.at[pl.ds(sid * RPT, RPT)])
    pltpu.sync_copy(ones_hbm, ones_v)
    pltpu.sync_copy(dst_hbm.at[pl.ds(wid * NCHUNK, NCHUNK)], dst_v)
    plsc.subcore_barrier()

    def body(j, carry):
        pltpu.sync_copy(ones_v, hist_sh.at[dst_v.at[j]], add=True)
        return carry

    lax.fori_loop(0, NCHUNK, body, 0)
    plsc.subcore_barrier()
    pltpu.sync_copy(hist_sh.at[pl.ds(sid * RPT, RPT)],
                    hist_hbm.at[cid].at[pl.ds(sid * RPT, RPT)])


@functools.lru_cache(maxsize=None)
def _sc_degree():
    mesh = plsc.VectorSubcoreMesh(core_axis_name="c", subcore_axis_name="s",
                                  num_cores=NC, num_subcores=NS)
    return pl.kernel(
        _sc_degree_body,
        out_type=jax.ShapeDtypeStruct((NC, NP), jnp.float32),
        mesh=mesh,
        scratch_types=[
            pltpu.VMEM((NCHUNK, C), jnp.int32),
            pltpu.VMEM((C,), jnp.float32),
            pltpu.VMEM_SHARED((NP,), jnp.float32),
            pltpu.SemaphoreType.DMA,
        ],
    )


NBUF = 4
LOOK = 2
BS = 8
SB = 3
DB = 4
Q0 = NCHUNK
Q1 = 2 * NCHUNK - Q0


def _sc_agg_body(x_hbm, src_hbm, dst_hbm, zeros_hbm, out_hbm,
                 srcb_v, dstb_v, rows_v, acc_sh, *sems):
    gsem = sems[:NBUF]
    ssem = sems[NBUF:]
    cid = lax.axis_index("c")
    sid = lax.axis_index("s")
    qc = lax.select(cid == 0, Q0, Q1)
    base = lax.select(cid == 0, sid * Q0, NS * Q0 + sid * Q1)
    pltpu.sync_copy(zeros_hbm.at[pl.ds(sid * RPT, RPT)],
                    acc_sh.at[pl.ds(sid * RPT, RPT)])
    for b in range(2):
        pltpu.sync_copy(src_hbm.at[pl.ds(base + b * BS, BS)],
                        srcb_v.at[b])
        pltpu.sync_copy(dst_hbm.at[pl.ds(base + b * BS, BS)],
                        dstb_v.at[b])
    plsc.subcore_barrier()

    for k in range(LOOK):
        pltpu.async_copy(x_hbm.at[srcb_v.at[0].at[k]], rows_v.at[k], gsem[k])

    def body(g, carry):
        for b in range(NBUF):
            j = g * NBUF + b
            blk = lax.div(j, BS)
            row = lax.rem(j, BS)
            pltpu.make_async_copy(
                x_hbm.at[srcb_v.at[lax.rem(blk, SB)].at[row]],
                rows_v.at[b], gsem[b]).wait()
            pltpu.async_copy(rows_v.at[b],
                             acc_sh.at[dstb_v.at[lax.rem(blk, DB)].at[row]],
                             ssem[b], add=True)

            @pl.when((row == 0) & ((blk + 2) * BS < qc))
            def _():
                pltpu.sync_copy(
                    src_hbm.at[pl.ds(base + (blk + 2) * BS, BS)],
                    srcb_v.at[lax.rem(blk + 2, SB)])
                pltpu.sync_copy(
                    dst_hbm.at[pl.ds(base + (blk + 2) * BS, BS)],
                    dstb_v.at[lax.rem(blk + 2, DB)])

            bn = (b + LOOK) % NBUF
            jn = j + LOOK

            @pl.when(jn < qc)
            def _():
                @pl.when(jn >= NBUF)
                def _():
                    pltpu.make_async_copy(rows_v.at[bn],
                                          acc_sh.at[dstb_v.at[0].at[0]],
                                          ssem[bn]).wait()

                pltpu.async_copy(
                    x_hbm.at[srcb_v.at[lax.rem(lax.div(jn, BS), SB)]
                             .at[lax.rem(jn, BS)]],
                    rows_v.at[bn], gsem[bn])

        return carry

    lax.fori_loop(0, lax.div(qc, NBUF), body, 0)
    for b in range(NBUF):
        pltpu.make_async_copy(rows_v.at[b], acc_sh.at[dstb_v.at[0].at[0]],
                              ssem[b]).wait()
    plsc.subcore_barrier()
    pltpu.sync_copy(acc_sh.at[pl.ds(sid * RPT, RPT)],
                    out_hbm.at[cid].at[pl.ds(sid * RPT, RPT)])


@functools.lru_cache(maxsize=None)
def _sc_agg():
    mesh = plsc.VectorSubcoreMesh(core_axis_name="c", subcore_axis_name="s",
                                  num_cores=NC, num_subcores=NS)
    return pl.kernel(
        _sc_agg_body,
        out_type=jax.ShapeDtypeStruct((NC, NP, D), jnp.float32),
        mesh=mesh,
        scratch_types=[
            pltpu.VMEM((SB, BS, C), jnp.int32),
            pltpu.VMEM((DB, BS, C), jnp.int32),
            pltpu.VMEM((NBUF, C, D), jnp.float32),
            pltpu.VMEM_SHARED((NP, D), jnp.float32),
        ] + [pltpu.SemaphoreType.DMA] * (2 * NBUF),
    )


def _norm_from_hist(hist):
    deg = hist[0] + hist[1]
    return lax.rsqrt(jnp.maximum(deg, 1.0))


def _tc_prep_body(hist_ref, x_ref, x1_ref):
    norm = _norm_from_hist(hist_ref[...])
    x1_ref[...] = x_ref[...] * norm[:, None]


def _tc_mid_body(hist_ref, agg_ref, x2_ref):
    norm = _norm_from_hist(hist_ref[...])
    agg = agg_ref[0] + agg_ref[1]
    x2_ref[...] = agg * (norm * norm)[:, None]


def _tc_final_body(hist_ref, agg_ref, w_ref, b_ref, out_ref):
    norm = _norm_from_hist(hist_ref[...])
    feat = (agg_ref[0] + agg_ref[1]) * norm[:, None]
    out_ref[...] = jnp.dot(feat, w_ref[...],
                           preferred_element_type=jnp.float32) + b_ref[...]


def kernel(in_feat, edge_index, W, b):
    npad = EPAD - E
    padsrc = jnp.arange(npad, dtype=jnp.int32) % N
    src = jnp.concatenate(
        [edge_index[0], padsrc]).reshape(NW * NCHUNK, C)
    trash = N + (jnp.arange(npad, dtype=jnp.int32) % (NP - N))
    dst = jnp.concatenate([edge_index[1], trash]).reshape(NW * NCHUNK, C)

    x = jnp.pad(in_feat, ((0, NP - N), (0, 0)))
    zerosn = jnp.zeros((NP,), jnp.float32)
    ones = jnp.ones((C,), jnp.float32)
    zeros = jnp.zeros((NP, D), jnp.float32)

    hist = _sc_degree()(dst, zerosn, ones)

    x1 = pl.pallas_call(
        _tc_prep_body,
        out_shape=jax.ShapeDtypeStruct((NP, D), jnp.float32),
    )(hist, x)

    agg1 = _sc_agg()(x1, src, dst, zeros)

    x2 = pl.pallas_call(
        _tc_mid_body,
        out_shape=jax.ShapeDtypeStruct((NP, D), jnp.float32),
    )(hist, agg1)

    agg2 = _sc_agg()(x2, src, dst, zeros)

    out = pl.pallas_call(
        _tc_final_body,
        out_shape=jax.ShapeDtypeStruct((NP, D), jnp.float32),
    )(hist, agg2, W, b.reshape(1, D))

    return out[:N]
